# baseline (speedup 1.0000x reference)
import functools
import math

import jax
import jax.numpy as jnp
from jax import lax
from jax.experimental import pallas as pl
from jax.experimental.pallas import tpu as pltpu
from jax.experimental.pallas import tpu_sc as plsc

VOC = 1000000
MAX_SEQ = 200
DIM = 64
SEQ = 200
BATCH = 4096
NC, NS, L = 2, 16, 16
NW = NC * NS
CHUNK = 128
NCH = SEQ * BATCH // CHUNK
PER_W = NCH // NW
NBUF = 4
CTILE = BATCH // CHUNK
EPS = 1e-5
SCALE = math.sqrt(DIM)
NJ = DIM // L


def _rsqrt(v):
    i = lax.bitcast_convert_type(v, jnp.int32)
    i = jnp.int32(0x5F3759DF) - (i >> 1)
    y = lax.bitcast_convert_type(i, jnp.float32)
    for _ in range(3):
        y = y * (1.5 - 0.5 * v * y * y)
    return y


def _body(item_hbm, pos_hbm, idxi_hbm, idxp_hbm, w_hbm, b_hbm, out_hbm,
          idxi_v, idxp_v, rows_v, pos_v, tile_v, w_v, b_v, gsem, wsem):
    wid = lax.axis_index("s") * NC + lax.axis_index("c")
    q0 = wid * PER_W

    pltpu.sync_copy(w_hbm, w_v)
    pltpu.sync_copy(b_hbm, b_v)
    wv = [w_v[pl.ds(j * L, L)] for j in range(NJ)]
    bv = [b_v[pl.ds(j * L, L)] for j in range(NJ)]
    lanes = lax.iota(jnp.int32, L)
    dts = [(jnp.full((L,), j * L, jnp.int32) + lanes) >> 3 for j in range(NJ)]
    drs = [(jnp.full((L,), j * L, jnp.int32) + lanes) & 7 for j in range(NJ)]

    def gather_descs(k):
        return (
            pltpu.make_async_copy(item_hbm.at[idxi_v.at[k]],
                                  rows_v.at[k], gsem),
            pltpu.make_async_copy(pos_hbm.at[idxp_v.at[k]],
                                  pos_v.at[k], gsem),
        )

    def issue(q, k):
        pltpu.sync_copy(idxi_hbm.at[pl.ds(q, 1)], idxi_v.at[pl.ds(k, 1)])
        pltpu.sync_copy(idxp_hbm.at[pl.ds(q, 1)], idxp_v.at[pl.ds(k, 1)])
        for d in gather_descs(k):
            d.start()

    def wb_desc(q, k):
        s = q // CTILE
        c = q % CTILE
        return pltpu.make_async_copy(tile_v.at[k], out_hbm.at[s, :, c], wsem)

    def compute(q, k):
        tile = tile_v.at[k]

        def token(t, _):
            x = [rows_v[k, t, pl.ds(j * L, L)] * SCALE
                 + pos_v[k, t, pl.ds(j * L, L)] for j in range(NJ)]
            tot = jnp.sum((x[0] + x[1]) + (x[2] + x[3]))
            sq = [xj * xj for xj in x]
            totsq = jnp.sum((sq[0] + sq[1]) + (sq[2] + sq[3]))
            mean = tot * (1.0 / DIM)
            var = jnp.maximum(totsq * (1.0 / DIM) - mean * mean, 0.0)
            rstd = _rsqrt(var + EPS)
            c = mean * rstd
            tsplat = jnp.full((L,), t, jnp.int32)
            for j in range(NJ):
                n = x[j] * rstd - c
                plsc.store_scatter(tile, [dts[j], drs[j], tsplat],
                                   n * wv[j] + bv[j])
            return _

        lax.fori_loop(0, CHUNK, token, None, unroll=4)

    issue(q0 + 0, 0)
    issue(q0 + 1, 1)

    def outer(i, _):
        for k in range(NBUF):
            j = i * NBUF + k
            q = q0 + j

            @pl.when(j + 2 < PER_W)
            def _issue_next():
                issue(q + 2, (k + 2) % NBUF)

            @pl.when(j >= NBUF)
            def _wait_wb():
                wb_desc(q, k).wait()

            for d in gather_descs(k):
                d.wait()
            compute(q, k)
            wb_desc(q, k).start()
        return _

    lax.fori_loop(0, PER_W // NBUF, outer, None)

    for k in range(NBUF):
        wb_desc(q0, k).wait()


@jax.jit
def _run(input_sequence, position_ids, item_table, pos_table, ln_weight,
         ln_bias):
    idxi = input_sequence.reshape(NCH, CHUNK)
    idxp = position_ids.reshape(NCH, CHUNK)
    mesh = plsc.VectorSubcoreMesh(core_axis_name="c", subcore_axis_name="s")
    k = pl.kernel(
        _body,
        out_type=jax.ShapeDtypeStruct((SEQ, DIM // 8, CTILE, 8, CHUNK),
                                      jnp.float32),
        mesh=mesh,
        scratch_types=[
            pltpu.VMEM((NBUF, CHUNK), jnp.int32),
            pltpu.VMEM((NBUF, CHUNK), jnp.int32),
            pltpu.VMEM((NBUF, CHUNK, DIM), jnp.float32),
            pltpu.VMEM((NBUF, CHUNK, DIM), jnp.float32),
            pltpu.VMEM((NBUF, DIM // 8, 8, CHUNK), jnp.float32),
            pltpu.VMEM((DIM,), jnp.float32),
            pltpu.VMEM((DIM,), jnp.float32),
            pltpu.SemaphoreType.DMA,
            pltpu.SemaphoreType.DMA,
        ],
        compiler_params=pltpu.CompilerParams(use_tc_tiling_on_sc=False,
                                             needs_layout_passes=False),
    )
    out5 = k(item_table, pos_table, idxi, idxp, ln_weight, ln_bias)
    return out5.transpose(0, 2, 4, 1, 3).reshape(SEQ, BATCH, DIM)


def kernel(input_sequence, position_ids, item_table, pos_table, ln_weight,
           ln_bias):
    return _run(input_sequence, position_ids, item_table, pos_table,
                ln_weight, ln_bias)


# speedup vs baseline: 1.4962x; 1.4962x over previous
import functools
import math

import jax
import jax.numpy as jnp
from jax import lax
from jax.experimental import pallas as pl
from jax.experimental.pallas import tpu as pltpu
from jax.experimental.pallas import tpu_sc as plsc

VOC = 1000000
MAX_SEQ = 200
DIM = 64
SEQ = 200
BATCH = 4096
N_TOK = SEQ * BATCH
NC, NS, L = 2, 16, 16
NW = NC * NS
PER_W = N_TOK // NW
CHUNK = 128
NBUF = 4
STEPS = PER_W // CHUNK
EPS = 1e-5
SCALE = math.sqrt(DIM)
NJ = DIM // L


def _rsqrt(v):
    i = lax.bitcast_convert_type(v, jnp.int32)
    i = jnp.full((L,), 0x5F3759DF, jnp.int32) - (i >> 1)
    y = lax.bitcast_convert_type(i, jnp.float32)
    for _ in range(2):
        y = y * (1.5 - 0.5 * v * y * y)
    return y


_DNUMS = lax.GatherDimensionNumbers(
    offset_dims=(), collapsed_slice_dims=(0,), start_index_map=(0,))


def _hsum(v, perms):
    for p in perms:
        v = v + lax.gather(v, p[:, None], _DNUMS, (1,),
                           mode=lax.GatherScatterMode.PROMISE_IN_BOUNDS)
    return v


def _body(item_hbm, pos_hbm, idxi_hbm, idxp_hbm, w_hbm, b_hbm, out_hbm,
          idxi_v, idxp_v, rows_v, pos_v, w_v, b_v, gsem, wsem):
    wid = lax.axis_index("s") * NC + lax.axis_index("c")
    row0 = wid * STEPS
    tok0 = wid * PER_W

    pltpu.sync_copy(w_hbm, w_v)
    pltpu.sync_copy(b_hbm, b_v)
    wv = [w_v[pl.ds(j * L, L)] for j in range(NJ)]
    bv = [b_v[pl.ds(j * L, L)] for j in range(NJ)]
    lanes = lax.iota(jnp.int32, L)
    perms = [lanes ^ k for k in (8, 4, 2, 1)]

    def gather_pair(s, k):
        return (
            pltpu.make_async_copy(item_hbm.at[idxi_v.at[k]],
                                  rows_v.at[pl.ds(k * CHUNK, CHUNK)], gsem),
            pltpu.make_async_copy(pos_hbm.at[idxp_v.at[k]],
                                  pos_v.at[pl.ds(k * CHUNK, CHUNK)], gsem),
        )

    def issue(s, k):
        pltpu.sync_copy(idxi_hbm.at[pl.ds(row0 + s, 1)],
                        idxi_v.at[pl.ds(k, 1)])
        pltpu.sync_copy(idxp_hbm.at[pl.ds(row0 + s, 1)],
                        idxp_v.at[pl.ds(k, 1)])
        for d in gather_pair(s, k):
            d.start()

    def wb_desc(s, k):
        return pltpu.make_async_copy(
            rows_v.at[pl.ds(k * CHUNK, CHUNK)],
            out_hbm.at[pl.ds(tok0 + s * CHUNK, CHUNK)], wsem)

    def compute(s, k):
        base = k * CHUNK

        def token(t, _):
            r = base + t
            x = [rows_v[r, pl.ds(j * L, L)] * SCALE + pos_v[r, pl.ds(j * L, L)]
                 for j in range(NJ)]
            tot = _hsum((x[0] + x[1]) + (x[2] + x[3]), perms)
            mean = tot * (1.0 / DIM)
            sq = [xj * xj for xj in x]
            sumsq = _hsum((sq[0] + sq[1]) + (sq[2] + sq[3]), perms)
            var = jnp.maximum(sumsq * (1.0 / DIM) - mean * mean, 0.0)
            rstd = _rsqrt(var + EPS)
            c = mean * rstd
            for j in range(NJ):
                n = x[j] * rstd - c
                rows_v[r, pl.ds(j * L, L)] = n * wv[j] + bv[j]
            return _

        lax.fori_loop(0, CHUNK, token, None, unroll=4)

    issue(0, 0)
    issue(1, 1)

    def outer(i, _):
        for k in range(NBUF):
            s = i * NBUF + k
            for d in gather_pair(s, k):
                d.wait()
            compute(s, k)
            wb_desc(s, k).start()
            kn = (k + 2) % NBUF

            @pl.when(s >= 2)
            def _wait_wb():
                wb_desc(s, kn).wait()

            @pl.when(s + 2 < STEPS)
            def _issue_next():
                issue(s + 2, kn)
        return _

    lax.fori_loop(0, STEPS // NBUF, outer, None)

    for k in range(2):
        wb_desc(0, k).wait()


@jax.jit
def _run(input_sequence, position_ids, item_table, pos_table, ln_weight,
         ln_bias):
    idxi = input_sequence.reshape(N_TOK // CHUNK, CHUNK)
    idxp = position_ids.reshape(N_TOK // CHUNK, CHUNK)
    mesh = plsc.VectorSubcoreMesh(core_axis_name="c", subcore_axis_name="s")
    k = pl.kernel(
        _body,
        out_type=jax.ShapeDtypeStruct((N_TOK, DIM), jnp.float32),
        mesh=mesh,
        scratch_types=[
            pltpu.VMEM((NBUF, CHUNK), jnp.int32),
            pltpu.VMEM((NBUF, CHUNK), jnp.int32),
            pltpu.VMEM((NBUF * CHUNK, DIM), jnp.float32),
            pltpu.VMEM((NBUF * CHUNK, DIM), jnp.float32),
            pltpu.VMEM((DIM,), jnp.float32),
            pltpu.VMEM((DIM,), jnp.float32),
            pltpu.SemaphoreType.DMA,
            pltpu.SemaphoreType.DMA,
        ],
        compiler_params=pltpu.CompilerParams(use_tc_tiling_on_sc=False, needs_layout_passes=False),
    )
    out = k(item_table, pos_table, idxi, idxp, ln_weight, ln_bias)
    return out.reshape(SEQ, BATCH, DIM)


def kernel(input_sequence, position_ids, item_table, pos_table, ln_weight,
           ln_bias):
    return _run(input_sequence, position_ids, item_table, pos_table,
                ln_weight, ln_bias)
